# baseline (speedup 1.0000x reference)
import jax
import jax.numpy as jnp
from jax.experimental import pallas as pl
from jax.experimental.pallas import tpu as pltpu

_TOPK = 100
_SCALE = 4.0
_NEG = -1.0
_BIG = 1 << 30
_HI = jax.lax.Precision.HIGHEST


def _t(colv):
    n = colv.shape[0]
    rowk = jax.lax.broadcasted_iota(jnp.int32, (n, n), 0)
    colj = jax.lax.broadcasted_iota(jnp.int32, (n, n), 1)
    z = jnp.zeros((), colv.dtype)
    return jnp.sum(jnp.where(rowk == colj, colv, z), axis=0, keepdims=True)


def _decode_kernel(hm_ref, off_ref, wh_ref, ids_ref, sc_ref, bb_ref,
                   s_ref, ma_ref, ra_ref, wa_ref, mb_ref, rb_ref, wb_ref):
    C, H, W = s_ref.shape
    CH = C // 2

    CHUNK = 8
    row_maxes = []
    for c0 in range(0, C, CHUNK):
        hm = hm_ref[0, c0:c0 + CHUNK, :, :]
        neg_col = jnp.full((CHUNK, H, 1), _NEG, jnp.float32)
        hl = jnp.concatenate([neg_col, hm[:, :, :W - 1]], axis=2)
        hr = jnp.concatenate([hm[:, :, 1:], neg_col], axis=2)
        wm = jnp.maximum(jnp.maximum(hl, hr), hm)
        neg_row = jnp.full((CHUNK, 1, W), _NEG, jnp.float32)
        hu = jnp.concatenate([neg_row, wm[:, :H - 1, :]], axis=1)
        hd = jnp.concatenate([wm[:, 1:, :], neg_row], axis=1)
        pool = jnp.maximum(jnp.maximum(hu, hd), wm)
        s = jnp.where(pool == hm, hm, 0.0)
        s_ref[c0:c0 + CHUNK, :, :] = s
        row_maxes.append(jnp.max(s, axis=2))
    rv0 = jnp.concatenate(row_maxes, axis=0)
    rvA0 = rv0[:CH, :]
    rvB0 = rv0[CH:, :]

    chioA = (jax.lax.broadcasted_iota(jnp.int32, (CH, H), 0) * H
             + jax.lax.broadcasted_iota(jnp.int32, (CH, H), 1))
    chioB = chioA + CH * H
    laneW3 = jax.lax.broadcasted_iota(jnp.int32, (1, 1, W), 2)

    def body(i, carry):
        rvA, rvB = carry

        mA = jnp.max(rvA)
        rA = jnp.min(jnp.where(rvA == mA, chioA, _BIG))
        cA = rA // H
        hA = rA % H
        srowA = s_ref[pl.ds(cA, 1), pl.ds(hA, 1), :]
        wA = jnp.min(jnp.where(srowA == mA, laneW3, _BIG))
        srowA2 = jnp.where(laneW3 == wA, _NEG, srowA)
        s_ref[pl.ds(cA, 1), pl.ds(hA, 1), :] = srowA2
        rvA2 = jnp.where(chioA == rA, jnp.max(srowA2), rvA)

        mB = jnp.max(rvB)
        rB = jnp.min(jnp.where(rvB == mB, chioB, _BIG))
        cB = rB // H
        hB = rB % H
        srowB = s_ref[pl.ds(cB, 1), pl.ds(hB, 1), :]
        wB = jnp.min(jnp.where(srowB == mB, laneW3, _BIG))
        srowB2 = jnp.where(laneW3 == wB, _NEG, srowB)
        s_ref[pl.ds(cB, 1), pl.ds(hB, 1), :] = srowB2
        rvB2 = jnp.where(chioB == rB, jnp.max(srowB2), rvB)

        ma_ref[pl.ds(i, 1), :] = jnp.full((1, 1), mA, jnp.float32)
        ra_ref[pl.ds(i, 1), :] = jnp.full((1, 1), rA, jnp.int32)
        wa_ref[pl.ds(i, 1), :] = jnp.full((1, 1), wA, jnp.int32)
        mb_ref[pl.ds(i, 1), :] = jnp.full((1, 1), mB, jnp.float32)
        rb_ref[pl.ds(i, 1), :] = jnp.full((1, 1), rB, jnp.int32)
        wb_ref[pl.ds(i, 1), :] = jnp.full((1, 1), wB, jnp.int32)
        return (rvA2, rvB2)

    jax.lax.fori_loop(0, _TOPK, body, (rvA0, rvB0))

    K = _TOPK
    mA_c = ma_ref[:, :]
    rA_c = ra_ref[:, :]
    wA_c = wa_ref[:, :]
    mB_c = mb_ref[:, :]
    rB_c = rb_ref[:, :]
    wB_c = wb_ref[:, :]

    mA_r = _t(mA_c)
    rA_r = _t(rA_c)
    mB_r = _t(mB_c)
    rB_r = _t(rB_c)

    T = ((mB_r > mA_c) | ((mB_r == mA_c) & (rB_r < rA_c))).astype(jnp.float32)
    Tp = ((mA_r > mB_c) | ((mA_r == mB_c) & (rA_r < rB_c))).astype(jnp.float32)

    kcol = jax.lax.broadcasted_iota(jnp.int32, (K, 1), 0)
    rankA = kcol + jnp.sum(T, axis=1, keepdims=True).astype(jnp.int32)
    rankB = kcol + jnp.sum(Tp, axis=1, keepdims=True).astype(jnp.int32)

    rank_all = jnp.concatenate([rankA, rankB], axis=0)
    m_all = jnp.concatenate([mA_c, mB_c], axis=0)
    r_all = jnp.concatenate([rA_c, rB_c], axis=0).astype(jnp.float32)
    w_all = jnp.concatenate([wA_c, wB_c], axis=0).astype(jnp.float32)

    rank_row = _t(rank_all)
    prow = jax.lax.broadcasted_iota(jnp.int32, (K, 2 * K), 0)
    O = jnp.where(rank_row == prow, 1.0, 0.0)
    mcol = jnp.dot(O, m_all, precision=_HI)
    rcolf = jnp.dot(O, r_all, precision=_HI)
    wcolf = jnp.dot(O, w_all, precision=_HI)
    rcol = rcolf.astype(jnp.int32)
    wcol = wcolf.astype(jnp.int32)
    ccol = rcol // H
    hcol = rcol % H

    colj = jax.lax.broadcasted_iota(jnp.int32, (K, W), 1)
    ohh = jnp.where(hcol == colj, 1.0, 0.0)
    ohw = jnp.where(wcol == colj, 1.0, 0.0)

    def gather_plane(plane):
        rows = jnp.dot(ohh, plane, precision=_HI,
                       preferred_element_type=jnp.float32)
        return jnp.sum(rows * ohw, axis=1, keepdims=True)

    ox = gather_plane(off_ref[0, 0, :, :])
    oy = gather_plane(off_ref[0, 1, :, :])
    bw = gather_plane(wh_ref[0, 0, :, :])
    bh = gather_plane(wh_ref[0, 1, :, :])

    xs = wcol.astype(jnp.float32) + ox
    ys = hcol.astype(jnp.float32) + oy
    hw = bw * 0.5
    hh = bh * 0.5
    x1 = (xs - hw) * _SCALE
    y1 = (ys - hh) * _SCALE
    x2 = (xs + hw) * _SCALE
    y2 = (ys + hh) * _SCALE

    sc_ref[0, :, :] = mcol
    ids_ref[0, :, :] = ccol.astype(jnp.float32)
    bb_ref[0, :, :] = jnp.concatenate([x1, y1, x2, y2], axis=1)


def _build_call(B, C, H, W, interpret=False):
    return pl.pallas_call(
        _decode_kernel,
        grid=(B,),
        in_specs=[
            pl.BlockSpec((1, C, H, W), lambda b: (b, 0, 0, 0)),
            pl.BlockSpec((1, 2, H, W), lambda b: (b, 0, 0, 0)),
            pl.BlockSpec((1, 2, H, W), lambda b: (b, 0, 0, 0)),
        ],
        out_specs=[
            pl.BlockSpec((1, _TOPK, 1), lambda b: (b, 0, 0)),
            pl.BlockSpec((1, _TOPK, 1), lambda b: (b, 0, 0)),
            pl.BlockSpec((1, _TOPK, 4), lambda b: (b, 0, 0)),
        ],
        out_shape=[
            jax.ShapeDtypeStruct((B, _TOPK, 1), jnp.float32),
            jax.ShapeDtypeStruct((B, _TOPK, 1), jnp.float32),
            jax.ShapeDtypeStruct((B, _TOPK, 4), jnp.float32),
        ],
        scratch_shapes=[
            pltpu.VMEM((C, H, W), jnp.float32),
            pltpu.VMEM((_TOPK, 1), jnp.float32),
            pltpu.VMEM((_TOPK, 1), jnp.int32),
            pltpu.VMEM((_TOPK, 1), jnp.int32),
            pltpu.VMEM((_TOPK, 1), jnp.float32),
            pltpu.VMEM((_TOPK, 1), jnp.int32),
            pltpu.VMEM((_TOPK, 1), jnp.int32),
        ],
        compiler_params=pltpu.CompilerParams(
            dimension_semantics=("parallel",)),
        interpret=interpret,
    )


@jax.jit
def kernel(heatmap, offset, wh):
    B, C, H, W = heatmap.shape
    ids, scores, bboxes = _build_call(B, C, H, W)(heatmap, offset, wh)
    return ids, scores, bboxes


# speedup vs baseline: 1.7359x; 1.7359x over previous
import jax
import jax.numpy as jnp
from jax.experimental import pallas as pl
from jax.experimental.pallas import tpu as pltpu

_TOPK = 100
_SCALE = 4.0
_NEG = -1.0
_BIG = 1 << 30


def _decode_kernel(hm_ref, off_ref, wh_ref, ids_ref, sc_ref, bb_ref,
                   s_ref, mcol_ref, rcol_ref, wcol_ref):
    C, H, W = s_ref.shape

    CHUNK = 8
    row_maxes = []
    for c0 in range(0, C, CHUNK):
        hm = hm_ref[0, c0:c0 + CHUNK, :, :]
        neg_col = jnp.full((CHUNK, H, 1), _NEG, jnp.float32)
        hl = jnp.concatenate([neg_col, hm[:, :, :W - 1]], axis=2)
        hr = jnp.concatenate([hm[:, :, 1:], neg_col], axis=2)
        wm = jnp.maximum(jnp.maximum(hl, hr), hm)
        neg_row = jnp.full((CHUNK, 1, W), _NEG, jnp.float32)
        hu = jnp.concatenate([neg_row, wm[:, :H - 1, :]], axis=1)
        hd = jnp.concatenate([wm[:, 1:, :], neg_row], axis=1)
        pool = jnp.maximum(jnp.maximum(hu, hd), wm)
        s = jnp.where(pool == hm, hm, 0.0)
        s_ref[c0:c0 + CHUNK, :, :] = s
        row_maxes.append(jnp.max(s, axis=2))
    rv0 = jnp.concatenate(row_maxes, axis=0)

    chio = (jax.lax.broadcasted_iota(jnp.int32, (C, H), 0) * H
            + jax.lax.broadcasted_iota(jnp.int32, (C, H), 1))
    laneW3 = jax.lax.broadcasted_iota(jnp.int32, (1, 1, W), 2)

    def body(i, rv):
        m = jnp.max(rv)
        r = jnp.min(jnp.where(rv == m, chio, _BIG))
        c = r // H
        h = r % H
        srow = s_ref[pl.ds(c, 1), pl.ds(h, 1), :]
        w = jnp.min(jnp.where(srow == m, laneW3, _BIG))
        srow2 = jnp.where(laneW3 == w, _NEG, srow)
        s_ref[pl.ds(c, 1), pl.ds(h, 1), :] = srow2
        m2 = jnp.max(srow2)
        mcol_ref[pl.ds(i, 1), :] = jnp.full((1, 1), m, jnp.float32)
        rcol_ref[pl.ds(i, 1), :] = jnp.full((1, 1), r, jnp.int32)
        wcol_ref[pl.ds(i, 1), :] = jnp.full((1, 1), w, jnp.int32)
        return jnp.where(chio == r, m2, rv)

    jax.lax.fori_loop(0, _TOPK, body, rv0)

    mcol = mcol_ref[:, :]
    rcol = rcol_ref[:, :]
    wcol = wcol_ref[:, :]
    ccol = rcol // H
    hcol = rcol % H

    colj = jax.lax.broadcasted_iota(jnp.int32, (_TOPK, W), 1)
    ohh = jnp.where(hcol == colj, 1.0, 0.0)
    ohw = jnp.where(wcol == colj, 1.0, 0.0)

    def gather_plane(plane):
        rows = jnp.dot(ohh, plane, precision=jax.lax.Precision.HIGHEST,
                       preferred_element_type=jnp.float32)
        return jnp.sum(rows * ohw, axis=1, keepdims=True)

    ox = gather_plane(off_ref[0, 0, :, :])
    oy = gather_plane(off_ref[0, 1, :, :])
    bw = gather_plane(wh_ref[0, 0, :, :])
    bh = gather_plane(wh_ref[0, 1, :, :])

    xs = wcol.astype(jnp.float32) + ox
    ys = hcol.astype(jnp.float32) + oy
    hw = bw * 0.5
    hh = bh * 0.5
    x1 = (xs - hw) * _SCALE
    y1 = (ys - hh) * _SCALE
    x2 = (xs + hw) * _SCALE
    y2 = (ys + hh) * _SCALE

    sc_ref[0, :, :] = mcol
    ids_ref[0, :, :] = ccol.astype(jnp.float32)
    bb_ref[0, :, :] = jnp.concatenate([x1, y1, x2, y2], axis=1)


def _build_call(B, C, H, W, interpret=False):
    return pl.pallas_call(
        _decode_kernel,
        grid=(B,),
        in_specs=[
            pl.BlockSpec((1, C, H, W), lambda b: (b, 0, 0, 0)),
            pl.BlockSpec((1, 2, H, W), lambda b: (b, 0, 0, 0)),
            pl.BlockSpec((1, 2, H, W), lambda b: (b, 0, 0, 0)),
        ],
        out_specs=[
            pl.BlockSpec((1, _TOPK, 1), lambda b: (b, 0, 0)),
            pl.BlockSpec((1, _TOPK, 1), lambda b: (b, 0, 0)),
            pl.BlockSpec((1, _TOPK, 4), lambda b: (b, 0, 0)),
        ],
        out_shape=[
            jax.ShapeDtypeStruct((B, _TOPK, 1), jnp.float32),
            jax.ShapeDtypeStruct((B, _TOPK, 1), jnp.float32),
            jax.ShapeDtypeStruct((B, _TOPK, 4), jnp.float32),
        ],
        scratch_shapes=[
            pltpu.VMEM((C, H, W), jnp.float32),
            pltpu.VMEM((_TOPK, 1), jnp.float32),
            pltpu.VMEM((_TOPK, 1), jnp.int32),
            pltpu.VMEM((_TOPK, 1), jnp.int32),
        ],
        compiler_params=pltpu.CompilerParams(
            dimension_semantics=("parallel",)),
        interpret=interpret,
    )


@jax.jit
def kernel(heatmap, offset, wh):
    B, C, H, W = heatmap.shape
    ids, scores, bboxes = _build_call(B, C, H, W)(heatmap, offset, wh)
    return ids, scores, bboxes
